# baseline (speedup 1.0000x reference)
import functools

import jax
import jax.numpy as jnp
from jax import lax
from jax.experimental import pallas as pl
from jax.experimental.pallas import tpu as pltpu
from jax.experimental.pallas import tpu_sc as plsc

NC, NS = 2, 16
NW = NC * NS
B, S, E = 4096, 200, 64
V = 1_000_000
BPW = B // NW
S0 = 128
S1 = S - S0
NBUF = 4
LANES = 16


def _widen_body(in_ref, out_ref):
    out_ref[:, :E] = in_ref[...]


def _widen(table):
    rows = 8000
    grid = V // rows
    return pl.pallas_call(
        _widen_body,
        grid=(grid,),
        in_specs=[pl.BlockSpec((rows, E), lambda i: (i, 0))],
        out_specs=pl.BlockSpec((rows, 2 * E), lambda i: (i, 0)),
        out_shape=jax.ShapeDtypeStruct((V, 2 * E), jnp.float32),
    )(table)


def _pool_body(x_hbm, table_hbm, out_hbm, idx_v, bufs, rep_v, sems):
    wid = lax.axis_index("s") * NC + lax.axis_index("c")
    base = wid * BPW

    pltpu.sync_copy(x_hbm.at[pl.ds(base, BPW), :], idx_v)

    def start(e, buf, sem):
        pltpu.async_copy(table_hbm.at[idx_v.at[e, pl.ds(0, S0)]],
                         buf.at[pl.ds(0, S0), :], sem)
        pltpu.async_copy(table_hbm.at[idx_v.at[e, pl.ds(S0, S1)]],
                         buf.at[pl.ds(S0, S1), :], sem)

    def wait(e, buf, sem):
        pltpu.make_async_copy(table_hbm.at[idx_v.at[e, pl.ds(0, S0)]],
                              buf.at[pl.ds(0, S0), :], sem).wait()
        pltpu.make_async_copy(table_hbm.at[idx_v.at[e, pl.ds(S0, S1)]],
                              buf.at[pl.ds(S0, S1), :], sem).wait()

    def reduce_into(e, buf):
        def body(j, acc):
            a0, a1, a2, a3 = acc
            for k in range(4):
                r = j * 4 + k
                a0 = a0 + buf[r, pl.ds(0 * LANES, LANES)]
                a1 = a1 + buf[r, pl.ds(1 * LANES, LANES)]
                a2 = a2 + buf[r, pl.ds(2 * LANES, LANES)]
                a3 = a3 + buf[r, pl.ds(3 * LANES, LANES)]
            return a0, a1, a2, a3

        z = jnp.zeros((LANES,), jnp.float32)
        a0, a1, a2, a3 = lax.fori_loop(0, S // 4, body, (z, z, z, z))
        rep_v[e, pl.ds(0 * LANES, LANES)] = a0
        rep_v[e, pl.ds(1 * LANES, LANES)] = a1
        rep_v[e, pl.ds(2 * LANES, LANES)] = a2
        rep_v[e, pl.ds(3 * LANES, LANES)] = a3

    for k in range(NBUF):
        start(k, bufs[k], sems[k])

    def outer(g, carry):
        for k in range(NBUF):
            e = g * NBUF + k
            wait(e, bufs[k], sems[k])
            reduce_into(e, bufs[k])

            @pl.when(g < BPW // NBUF - 1)
            def _():
                start(e + NBUF, bufs[k], sems[k])
        return carry

    lax.fori_loop(0, BPW // NBUF, outer, 0)

    pltpu.sync_copy(rep_v, out_hbm.at[pl.ds(base, BPW), :])


def _pool(x_lin, table_lin):
    def body(x_hbm, table_hbm, out_hbm, idx_v, b0, b1, b2, b3, rep_v,
             s0, s1, s2, s3):
        _pool_body(x_hbm, table_hbm, out_hbm, idx_v,
                   (b0, b1, b2, b3), rep_v, (s0, s1, s2, s3))

    fn = pl.kernel(
        body,
        out_type=jax.ShapeDtypeStruct((B, E), jnp.float32),
        mesh=plsc.VectorSubcoreMesh(core_axis_name="c", subcore_axis_name="s"),
        scratch_types=[
            pltpu.VMEM((BPW, S), jnp.int32),
            pltpu.VMEM((S, E), jnp.float32),
            pltpu.VMEM((S, E), jnp.float32),
            pltpu.VMEM((S, E), jnp.float32),
            pltpu.VMEM((S, E), jnp.float32),
            pltpu.VMEM((BPW, E), jnp.float32),
            pltpu.SemaphoreType.DMA,
            pltpu.SemaphoreType.DMA,
            pltpu.SemaphoreType.DMA,
            pltpu.SemaphoreType.DMA,
        ],
        compiler_params=pltpu.CompilerParams(use_tc_tiling_on_sc=False),
    )
    return fn(x_lin, table_lin)


def _head_body(rep_ref, inv_ref, w1_ref, b1_ref, w2_ref, b2_ref, out_ref):
    rep = rep_ref[...] * inv_ref[...]
    h = jnp.dot(rep, w1_ref[...], preferred_element_type=jnp.float32)
    h = jnp.maximum(h + b1_ref[...], 0.0)
    out_ref[...] = (
        jnp.dot(h, w2_ref[...], preferred_element_type=jnp.float32)
        + b2_ref[...]
    )


def _head(rep, lengths, W1, b1, W2, b2):
    inv = (1.0 / lengths.astype(jnp.float32)).reshape(B, 1)
    bm = 512
    grid = B // bm
    return pl.pallas_call(
        _head_body,
        grid=(grid,),
        in_specs=[
            pl.BlockSpec((bm, E), lambda i: (i, 0)),
            pl.BlockSpec((bm, 1), lambda i: (i, 0)),
            pl.BlockSpec(W1.shape, lambda i: (0, 0)),
            pl.BlockSpec((1, b1.shape[0]), lambda i: (0, 0)),
            pl.BlockSpec(W2.shape, lambda i: (0, 0)),
            pl.BlockSpec((1, b2.shape[0]), lambda i: (0, 0)),
        ],
        out_specs=pl.BlockSpec((bm, b2.shape[0]), lambda i: (i, 0)),
        out_shape=jax.ShapeDtypeStruct((B, b2.shape[0]), jnp.float32),
    )(rep, inv, W1, b1.reshape(1, -1), W2, b2.reshape(1, -1))


@jax.jit
def kernel(x, lengths, table, W1, b1, W2, b2):
    table_lin = _widen(table).reshape(2 * V, E)
    x2 = x.astype(jnp.int32) * 2
    rep_sum = _pool(x2, table_lin)
    return _head(rep_sum, lengths, W1, b1, W2, b2)


# speedup vs baseline: 1.6949x; 1.6949x over previous
import functools

import jax
import jax.numpy as jnp
from jax import lax
from jax.experimental import pallas as pl
from jax.experimental.pallas import tpu as pltpu
from jax.experimental.pallas import tpu_sc as plsc

NC, NS = 2, 16
NW = NC * NS
B, S, E = 4096, 200, 64
V = 1_000_000
BPW = B // NW
S0 = 128
S1 = S - S0
NBUF = 4
LANES = 16


def _widen_body(in_ref, out_ref):
    out_ref[:, :E] = in_ref[...].T


def _widen(tableT):
    cols = 4096
    grid = (V + cols - 1) // cols
    return pl.pallas_call(
        _widen_body,
        grid=(grid,),
        in_specs=[pl.BlockSpec((E, cols), lambda i: (0, i))],
        out_specs=pl.BlockSpec((cols, 2 * E), lambda i: (i, 0)),
        out_shape=jax.ShapeDtypeStruct((V, 2 * E), jnp.float32),
    )(tableT)


def _pool_body(x_hbm, table_hbm, out_hbm, idx_v, bufs, rep_v, sems):
    wid = lax.axis_index("s") * NC + lax.axis_index("c")
    base = wid * BPW

    pltpu.sync_copy(x_hbm.at[pl.ds(base, BPW), :], idx_v)

    def start(e, buf, sem):
        pltpu.async_copy(table_hbm.at[idx_v.at[e, pl.ds(0, S0)]],
                         buf.at[pl.ds(0, S0), :], sem)
        pltpu.async_copy(table_hbm.at[idx_v.at[e, pl.ds(S0, S1)]],
                         buf.at[pl.ds(S0, S1), :], sem)

    def wait(e, buf, sem):
        pltpu.make_async_copy(table_hbm.at[idx_v.at[e, pl.ds(0, S0)]],
                              buf.at[pl.ds(0, S0), :], sem).wait()
        pltpu.make_async_copy(table_hbm.at[idx_v.at[e, pl.ds(S0, S1)]],
                              buf.at[pl.ds(S0, S1), :], sem).wait()

    def reduce_into(e, buf):
        def body(j, acc):
            a0, a1, a2, a3 = acc
            for k in range(4):
                r = j * 4 + k
                a0 = a0 + buf[r, pl.ds(0 * LANES, LANES)]
                a1 = a1 + buf[r, pl.ds(1 * LANES, LANES)]
                a2 = a2 + buf[r, pl.ds(2 * LANES, LANES)]
                a3 = a3 + buf[r, pl.ds(3 * LANES, LANES)]
            return a0, a1, a2, a3

        z = jnp.zeros((LANES,), jnp.float32)
        a0, a1, a2, a3 = lax.fori_loop(0, S // 4, body, (z, z, z, z))
        rep_v[e, pl.ds(0 * LANES, LANES)] = a0
        rep_v[e, pl.ds(1 * LANES, LANES)] = a1
        rep_v[e, pl.ds(2 * LANES, LANES)] = a2
        rep_v[e, pl.ds(3 * LANES, LANES)] = a3

    for k in range(NBUF):
        start(k, bufs[k], sems[k])

    def outer(g, carry):
        for k in range(NBUF):
            e = g * NBUF + k
            wait(e, bufs[k], sems[k])
            reduce_into(e, bufs[k])

            @pl.when(g < BPW // NBUF - 1)
            def _():
                start(e + NBUF, bufs[k], sems[k])
        return carry

    lax.fori_loop(0, BPW // NBUF, outer, 0)

    pltpu.sync_copy(rep_v, out_hbm.at[pl.ds(base, BPW), :])


def _pool(x_lin, table_lin):
    def body(x_hbm, table_hbm, out_hbm, idx_v, b0, b1, b2, b3, rep_v,
             s0, s1, s2, s3):
        _pool_body(x_hbm, table_hbm, out_hbm, idx_v,
                   (b0, b1, b2, b3), rep_v, (s0, s1, s2, s3))

    fn = pl.kernel(
        body,
        out_type=jax.ShapeDtypeStruct((B, E), jnp.float32),
        mesh=plsc.VectorSubcoreMesh(core_axis_name="c", subcore_axis_name="s"),
        scratch_types=[
            pltpu.VMEM((BPW, S), jnp.int32),
            pltpu.VMEM((S, E), jnp.float32),
            pltpu.VMEM((S, E), jnp.float32),
            pltpu.VMEM((S, E), jnp.float32),
            pltpu.VMEM((S, E), jnp.float32),
            pltpu.VMEM((BPW, E), jnp.float32),
            pltpu.SemaphoreType.DMA,
            pltpu.SemaphoreType.DMA,
            pltpu.SemaphoreType.DMA,
            pltpu.SemaphoreType.DMA,
        ],
        compiler_params=pltpu.CompilerParams(use_tc_tiling_on_sc=False),
    )
    return fn(x_lin, table_lin)


def _head_body(rep_ref, inv_ref, w1_ref, b1_ref, w2_ref, b2_ref, out_ref):
    rep = rep_ref[...] * inv_ref[...]
    h = jnp.dot(rep, w1_ref[...], preferred_element_type=jnp.float32)
    h = jnp.maximum(h + b1_ref[...], 0.0)
    out_ref[...] = (
        jnp.dot(h, w2_ref[...], preferred_element_type=jnp.float32)
        + b2_ref[...]
    )


def _head(rep, lengths, W1, b1, W2, b2):
    inv = (1.0 / lengths.astype(jnp.float32)).reshape(B, 1)
    bm = 512
    grid = B // bm
    return pl.pallas_call(
        _head_body,
        grid=(grid,),
        in_specs=[
            pl.BlockSpec((bm, E), lambda i: (i, 0)),
            pl.BlockSpec((bm, 1), lambda i: (i, 0)),
            pl.BlockSpec(W1.shape, lambda i: (0, 0)),
            pl.BlockSpec((1, b1.shape[0]), lambda i: (0, 0)),
            pl.BlockSpec(W2.shape, lambda i: (0, 0)),
            pl.BlockSpec((1, b2.shape[0]), lambda i: (0, 0)),
        ],
        out_specs=pl.BlockSpec((bm, b2.shape[0]), lambda i: (i, 0)),
        out_shape=jax.ShapeDtypeStruct((B, b2.shape[0]), jnp.float32),
    )(rep, inv, W1, b1.reshape(1, -1), W2, b2.reshape(1, -1))


@jax.jit
def kernel(x, lengths, table, W1, b1, W2, b2):
    table_lin = _widen(table.T).reshape(2 * V, E)
    x2 = x.astype(jnp.int32) * 2
    rep_sum = _pool(x2, table_lin)
    return _head(rep_sum, lengths, W1, b1, W2, b2)


# speedup vs baseline: 1.7482x; 1.0315x over previous
import functools

import jax
import jax.numpy as jnp
from jax import lax
from jax.experimental import pallas as pl
from jax.experimental.pallas import tpu as pltpu
from jax.experimental.pallas import tpu_sc as plsc

NC, NS = 2, 16
NW = NC * NS
B, S, E = 4096, 200, 64
V = 1_000_000
BPW = B // NW
S0 = 128
S1 = S - S0
NBUF = 4
LANES = 16


CB = 4096
NBLK = (V + CB - 1) // CB
VL = NBLK * CB


def _xpose_body(in_ref, out_ref):
    t = in_ref[...].T
    out_ref[:, :E] = t[: CB // 2]
    out_ref[:, E:] = t[CB // 2:]


def _xpose(tableT):
    return pl.pallas_call(
        _xpose_body,
        grid=(NBLK,),
        in_specs=[pl.BlockSpec((E, CB), lambda i: (0, i))],
        out_specs=pl.BlockSpec((CB // 2, 2 * E), lambda i: (i, 0)),
        out_shape=jax.ShapeDtypeStruct((NBLK * CB // 2, 2 * E), jnp.float32),
    )(tableT)


def _pool_body(x_hbm, table_hbm, out_hbm, idx_v, bufs, rep_v, sems):
    wid = lax.axis_index("s") * NC + lax.axis_index("c")
    base = wid * BPW

    pltpu.sync_copy(x_hbm.at[pl.ds(base, BPW), :], idx_v)

    def start(e, buf, sem):
        pltpu.async_copy(table_hbm.at[idx_v.at[e, pl.ds(0, S0)]],
                         buf.at[pl.ds(0, S0), :], sem)
        pltpu.async_copy(table_hbm.at[idx_v.at[e, pl.ds(S0, S1)]],
                         buf.at[pl.ds(S0, S1), :], sem)

    def wait(e, buf, sem):
        pltpu.make_async_copy(table_hbm.at[idx_v.at[e, pl.ds(0, S0)]],
                              buf.at[pl.ds(0, S0), :], sem).wait()
        pltpu.make_async_copy(table_hbm.at[idx_v.at[e, pl.ds(S0, S1)]],
                              buf.at[pl.ds(S0, S1), :], sem).wait()

    def reduce_into(e, buf):
        def body(j, acc):
            a0, a1, a2, a3 = acc
            for k in range(4):
                r = j * 4 + k
                a0 = a0 + buf[r, pl.ds(0 * LANES, LANES)]
                a1 = a1 + buf[r, pl.ds(1 * LANES, LANES)]
                a2 = a2 + buf[r, pl.ds(2 * LANES, LANES)]
                a3 = a3 + buf[r, pl.ds(3 * LANES, LANES)]
            return a0, a1, a2, a3

        z = jnp.zeros((LANES,), jnp.float32)
        a0, a1, a2, a3 = lax.fori_loop(0, S // 4, body, (z, z, z, z))
        rep_v[e, pl.ds(0 * LANES, LANES)] = a0
        rep_v[e, pl.ds(1 * LANES, LANES)] = a1
        rep_v[e, pl.ds(2 * LANES, LANES)] = a2
        rep_v[e, pl.ds(3 * LANES, LANES)] = a3

    for k in range(NBUF):
        start(k, bufs[k], sems[k])

    def outer(g, carry):
        for k in range(NBUF):
            e = g * NBUF + k
            wait(e, bufs[k], sems[k])
            reduce_into(e, bufs[k])

            @pl.when(g < BPW // NBUF - 1)
            def _():
                start(e + NBUF, bufs[k], sems[k])
        return carry

    lax.fori_loop(0, BPW // NBUF, outer, 0)

    pltpu.sync_copy(rep_v, out_hbm.at[pl.ds(base, BPW), :])


def _pool(x_lin, table_lin):
    def body(x_hbm, table_hbm, out_hbm, idx_v, b0, b1, b2, b3, rep_v,
             s0, s1, s2, s3):
        _pool_body(x_hbm, table_hbm, out_hbm, idx_v,
                   (b0, b1, b2, b3), rep_v, (s0, s1, s2, s3))

    fn = pl.kernel(
        body,
        out_type=jax.ShapeDtypeStruct((B, E), jnp.float32),
        mesh=plsc.VectorSubcoreMesh(core_axis_name="c", subcore_axis_name="s"),
        scratch_types=[
            pltpu.VMEM((BPW, S), jnp.int32),
            pltpu.VMEM((S, E), jnp.float32),
            pltpu.VMEM((S, E), jnp.float32),
            pltpu.VMEM((S, E), jnp.float32),
            pltpu.VMEM((S, E), jnp.float32),
            pltpu.VMEM((BPW, E), jnp.float32),
            pltpu.SemaphoreType.DMA,
            pltpu.SemaphoreType.DMA,
            pltpu.SemaphoreType.DMA,
            pltpu.SemaphoreType.DMA,
        ],
        compiler_params=pltpu.CompilerParams(use_tc_tiling_on_sc=False),
    )
    return fn(x_lin, table_lin)


def _head_body(rep_ref, inv_ref, w1_ref, b1_ref, w2_ref, b2_ref, out_ref):
    rep = rep_ref[...] * inv_ref[...]
    h = jnp.dot(rep, w1_ref[...], preferred_element_type=jnp.float32)
    h = jnp.maximum(h + b1_ref[...], 0.0)
    out_ref[...] = (
        jnp.dot(h, w2_ref[...], preferred_element_type=jnp.float32)
        + b2_ref[...]
    )


def _head(rep, lengths, W1, b1, W2, b2):
    inv = (1.0 / lengths.astype(jnp.float32)).reshape(B, 1)
    bm = 512
    grid = B // bm
    return pl.pallas_call(
        _head_body,
        grid=(grid,),
        in_specs=[
            pl.BlockSpec((bm, E), lambda i: (i, 0)),
            pl.BlockSpec((bm, 1), lambda i: (i, 0)),
            pl.BlockSpec(W1.shape, lambda i: (0, 0)),
            pl.BlockSpec((1, b1.shape[0]), lambda i: (0, 0)),
            pl.BlockSpec(W2.shape, lambda i: (0, 0)),
            pl.BlockSpec((1, b2.shape[0]), lambda i: (0, 0)),
        ],
        out_specs=pl.BlockSpec((bm, b2.shape[0]), lambda i: (i, 0)),
        out_shape=jax.ShapeDtypeStruct((B, b2.shape[0]), jnp.float32),
    )(rep, inv, W1, b1.reshape(1, -1), W2, b2.reshape(1, -1))


@jax.jit
def kernel(x, lengths, table, W1, b1, W2, b2):
    table_lin = _xpose(table.T).reshape(VL, E)
    xi = x.astype(jnp.int32)
    x2 = ((xi >> 12) << 12) + 2 * (xi & (CB // 2 - 1)) + ((xi >> 11) & 1)
    rep_sum = _pool(x2, table_lin)
    return _head(rep_sum, lengths, W1, b1, W2, b2)
